# baseline (speedup 1.0000x reference)
import functools

import jax
import jax.numpy as jnp
from jax import lax
from jax.experimental import pallas as pl
from jax.experimental.pallas import tpu as pltpu
from jax.experimental.pallas import tpu_sc as plsc

B = 16384
D = 32
NC = 2
NS = 16
NW = NC * NS
BPW = B // NW
CHUNK = 128
NCHUNK = BPW // CHUNK


@functools.lru_cache(maxsize=None)
def _make_sc_gather():
    mesh = plsc.VectorSubcoreMesh(
        core_axis_name="c", subcore_axis_name="s", num_cores=NC, num_subcores=NS
    )

    @functools.partial(
        pl.kernel,
        out_type=[jax.ShapeDtypeStruct((B, D), jnp.float32) for _ in range(4)],
        mesh=mesh,
        scratch_types=[
            pltpu.VMEM((NCHUNK, CHUNK), jnp.int32),
            pltpu.VMEM((NCHUNK, CHUNK), jnp.int32),
            pltpu.SemaphoreType.DMA,
        ],
    )
    def _sc_gather(uid_h, jid_h, umt_h, jmt_h, ugt_h, jgt_h,
                   out_um, out_jm, out_ug, out_jg,
                   uidx, jidx, sem):
        wid = lax.axis_index("s") * NC + lax.axis_index("c")
        r0 = wid * NCHUNK
        pltpu.sync_copy(uid_h.at[pl.ds(r0, NCHUNK)], uidx)
        pltpu.sync_copy(jid_h.at[pl.ds(r0, NCHUNK)], jidx)
        base = wid * BPW
        L = 16

        for c in range(NCHUNK):
            def issue(g, _, c=c):
                vu = uidx[c, pl.ds(g * L, L)]
                vj = jidx[c, pl.ds(g * L, L)]
                row0 = base + c * CHUNK + g * L
                for i in range(L):
                    u = vu[i]
                    j = vj[i]
                    g_row = row0 + i
                    pltpu.async_copy(umt_h.at[pl.ds(u, 1)], out_um.at[pl.ds(g_row, 1)], sem)
                    pltpu.async_copy(jmt_h.at[pl.ds(j, 1)], out_jm.at[pl.ds(g_row, 1)], sem)
                    pltpu.async_copy(ugt_h.at[pl.ds(u, 1)], out_ug.at[pl.ds(g_row, 1)], sem)
                    pltpu.async_copy(jgt_h.at[pl.ds(j, 1)], out_jg.at[pl.ds(g_row, 1)], sem)
                return ()
            lax.fori_loop(0, CHUNK // L, issue, ())

        def drain(k, _):
            pltpu.make_async_copy(umt_h.at[pl.ds(0, 1)], out_um.at[pl.ds(base, 1)], sem).wait()
            pltpu.make_async_copy(jmt_h.at[pl.ds(0, 1)], out_jm.at[pl.ds(base, 1)], sem).wait()
            pltpu.make_async_copy(ugt_h.at[pl.ds(0, 1)], out_ug.at[pl.ds(base, 1)], sem).wait()
            pltpu.make_async_copy(jgt_h.at[pl.ds(0, 1)], out_jg.at[pl.ds(base, 1)], sem).wait()
            return ()
        lax.fori_loop(0, BPW, drain, ())

    return _sc_gather


BLK = 2048


def _tc_body(um, jm, ug, jg, w1u, w1j, b1, w2, b2, w3, b3, w4, scal, out):
    x = jnp.maximum(um[:] @ w1u[:] + jm[:] @ w1j[:] + b1[:], 0.0)
    x = jnp.maximum(x @ w2[:] + b2[:], 0.0)
    x = jnp.maximum(x @ w3[:] + b3[:], 0.0)
    x = jnp.sum(x * w4[:], axis=1, keepdims=True) + scal[0]
    x = jnp.maximum(x, 0.0)
    u = ug[:]
    j = jg[:]
    dot = jnp.sum(u * j, axis=1, keepdims=True)
    su = jnp.sum(u * u, axis=1, keepdims=True)
    sj = jnp.sum(j * j, axis=1, keepdims=True)
    gmf = dot * lax.rsqrt(jnp.maximum(su, 1e-12)) * lax.rsqrt(jnp.maximum(sj, 1e-12))
    out[:] = x * scal[1] + gmf * scal[2] + scal[3]


def _tc_dense(um, jm, ug, jg, w1u, w1j, b1, w2, b2, w3, b3, w4, scal):
    row = pl.BlockSpec((BLK, D), lambda i: (i, 0))
    full = lambda a: pl.BlockSpec(a.shape, lambda i, _n=a.ndim: (0,) * _n)
    return pl.pallas_call(
        _tc_body,
        grid=(B // BLK,),
        in_specs=[row, row, row, row,
                  full(w1u), full(w1j), full(b1), full(w2), full(b2),
                  full(w3), full(b3), full(w4),
                  pl.BlockSpec(memory_space=pltpu.SMEM)],
        out_specs=pl.BlockSpec((BLK, 1), lambda i: (i, 0)),
        out_shape=jax.ShapeDtypeStruct((B, 1), jnp.float32),
    )(um, jm, ug, jg, w1u, w1j, b1, w2, b2, w3, b3, w4, scal)


def kernel(user_ids, joke_ids, user_mlp_table, joke_mlp_table,
           user_gmf_table, joke_gmf_table,
           W1, b1, W2, b2, W3, b3, W4, b4, W5, b5):
    uid = user_ids.astype(jnp.int32).reshape(B // CHUNK, CHUNK)
    jid = joke_ids.astype(jnp.int32).reshape(B // CHUNK, CHUNK)
    um, jm, ug, jg = _make_sc_gather()(uid, jid, user_mlp_table, joke_mlp_table,
                                       user_gmf_table, joke_gmf_table)
    w1u = W1[:D, :]
    w1j = W1[D:, :]
    scal = jnp.stack([b4[0], W5[0, 0], W5[1, 0], b5[0]])
    return _tc_dense(um, jm, ug, jg, w1u, w1j, b1.reshape(1, -1),
                     W2, b2.reshape(1, -1), W3, b3.reshape(1, -1),
                     W4.reshape(1, -1), scal)


# speedup vs baseline: 2.4181x; 2.4181x over previous
import functools

import jax
import jax.numpy as jnp
from jax import lax
from jax.experimental import pallas as pl
from jax.experimental.pallas import tpu as pltpu
from jax.experimental.pallas import tpu_sc as plsc

B = 16384
D = 32
NC = 2
NS = 16
NW = NC * NS
BPW = B // NW
CHUNK = 128
NCHUNK = BPW // CHUNK


@functools.lru_cache(maxsize=None)
def _make_sc_gather():
    mesh = plsc.VectorSubcoreMesh(
        core_axis_name="c", subcore_axis_name="s", num_cores=NC, num_subcores=NS
    )

    @functools.partial(
        pl.kernel,
        out_type=[jax.ShapeDtypeStruct((B, D), jnp.float32) for _ in range(4)],
        mesh=mesh,
        scratch_types=[
            pltpu.VMEM((NCHUNK, CHUNK), jnp.int32),
            pltpu.VMEM((NCHUNK, CHUNK), jnp.int32),
            pltpu.VMEM((CHUNK, D), jnp.float32),
            pltpu.VMEM((CHUNK, D), jnp.float32),
            pltpu.VMEM((CHUNK, D), jnp.float32),
            pltpu.VMEM((CHUNK, D), jnp.float32),
            pltpu.SemaphoreType.DMA,
            pltpu.SemaphoreType.DMA,
        ],
    )
    def _sc_gather(uid_h, jid_h, umt_h, jmt_h, ugt_h, jgt_h,
                   out_um, out_jm, out_ug, out_jg,
                   uidx, jidx, bum, bjm, bug, bjg, sem, wsem):
        wid = lax.axis_index("s") * NC + lax.axis_index("c")
        r0 = wid * NCHUNK
        pltpu.sync_copy(uid_h.at[pl.ds(r0, NCHUNK)], uidx)
        pltpu.sync_copy(jid_h.at[pl.ds(r0, NCHUNK)], jidx)
        base = wid * BPW
        L = 16

        for c in range(NCHUNK):
            if c > 0:
                pltpu.make_async_copy(bum, out_um.at[pl.ds(base, CHUNK)], wsem).wait()
                pltpu.make_async_copy(bjm, out_jm.at[pl.ds(base, CHUNK)], wsem).wait()
                pltpu.make_async_copy(bug, out_ug.at[pl.ds(base, CHUNK)], wsem).wait()
                pltpu.make_async_copy(bjg, out_jg.at[pl.ds(base, CHUNK)], wsem).wait()

            def issue(g, _, c=c):
                vu = uidx[c, pl.ds(g * L, L)]
                vj = jidx[c, pl.ds(g * L, L)]
                row0 = g * L
                for i in range(L):
                    u = vu[i]
                    j = vj[i]
                    k = row0 + i
                    pltpu.async_copy(umt_h.at[pl.ds(u, 1)], bum.at[pl.ds(k, 1)], sem)
                    pltpu.async_copy(jmt_h.at[pl.ds(j, 1)], bjm.at[pl.ds(k, 1)], sem)
                    pltpu.async_copy(ugt_h.at[pl.ds(u, 1)], bug.at[pl.ds(k, 1)], sem)
                    pltpu.async_copy(jgt_h.at[pl.ds(j, 1)], bjg.at[pl.ds(k, 1)], sem)
                return ()
            lax.fori_loop(0, CHUNK // L, issue, ())

            def drain(k, _):
                pltpu.make_async_copy(umt_h.at[pl.ds(0, 1)], bum.at[pl.ds(0, 1)], sem).wait()
                pltpu.make_async_copy(jmt_h.at[pl.ds(0, 1)], bjm.at[pl.ds(0, 1)], sem).wait()
                pltpu.make_async_copy(ugt_h.at[pl.ds(0, 1)], bug.at[pl.ds(0, 1)], sem).wait()
                pltpu.make_async_copy(jgt_h.at[pl.ds(0, 1)], bjg.at[pl.ds(0, 1)], sem).wait()
                return ()
            lax.fori_loop(0, CHUNK, drain, ())

            dst = pl.ds(base + c * CHUNK, CHUNK)
            pltpu.async_copy(bum, out_um.at[dst], wsem)
            pltpu.async_copy(bjm, out_jm.at[dst], wsem)
            pltpu.async_copy(bug, out_ug.at[dst], wsem)
            pltpu.async_copy(bjg, out_jg.at[dst], wsem)

        pltpu.make_async_copy(bum, out_um.at[pl.ds(base, CHUNK)], wsem).wait()
        pltpu.make_async_copy(bjm, out_jm.at[pl.ds(base, CHUNK)], wsem).wait()
        pltpu.make_async_copy(bug, out_ug.at[pl.ds(base, CHUNK)], wsem).wait()
        pltpu.make_async_copy(bjg, out_jg.at[pl.ds(base, CHUNK)], wsem).wait()

    return _sc_gather


BLK = 2048


def _tc_body(um, jm, ug, jg, w1u, w1j, b1, w2, b2, w3, b3, w4, scal, out):
    x = jnp.maximum(um[:] @ w1u[:] + jm[:] @ w1j[:] + b1[:], 0.0)
    x = jnp.maximum(x @ w2[:] + b2[:], 0.0)
    x = jnp.maximum(x @ w3[:] + b3[:], 0.0)
    x = jnp.sum(x * w4[:], axis=1, keepdims=True) + scal[0]
    x = jnp.maximum(x, 0.0)
    u = ug[:]
    j = jg[:]
    dot = jnp.sum(u * j, axis=1, keepdims=True)
    su = jnp.sum(u * u, axis=1, keepdims=True)
    sj = jnp.sum(j * j, axis=1, keepdims=True)
    gmf = dot * lax.rsqrt(jnp.maximum(su, 1e-12)) * lax.rsqrt(jnp.maximum(sj, 1e-12))
    out[:] = x * scal[1] + gmf * scal[2] + scal[3]


def _tc_dense(um, jm, ug, jg, w1u, w1j, b1, w2, b2, w3, b3, w4, scal):
    row = pl.BlockSpec((BLK, D), lambda i: (i, 0))
    full = lambda a: pl.BlockSpec(a.shape, lambda i, _n=a.ndim: (0,) * _n)
    return pl.pallas_call(
        _tc_body,
        grid=(B // BLK,),
        in_specs=[row, row, row, row,
                  full(w1u), full(w1j), full(b1), full(w2), full(b2),
                  full(w3), full(b3), full(w4),
                  pl.BlockSpec(memory_space=pltpu.SMEM)],
        out_specs=pl.BlockSpec((BLK, 1), lambda i: (i, 0)),
        out_shape=jax.ShapeDtypeStruct((B, 1), jnp.float32),
    )(um, jm, ug, jg, w1u, w1j, b1, w2, b2, w3, b3, w4, scal)


def kernel(user_ids, joke_ids, user_mlp_table, joke_mlp_table,
           user_gmf_table, joke_gmf_table,
           W1, b1, W2, b2, W3, b3, W4, b4, W5, b5):
    uid = user_ids.astype(jnp.int32).reshape(B // CHUNK, CHUNK)
    jid = joke_ids.astype(jnp.int32).reshape(B // CHUNK, CHUNK)
    um, jm, ug, jg = _make_sc_gather()(uid, jid, user_mlp_table, joke_mlp_table,
                                       user_gmf_table, joke_gmf_table)
    w1u = W1[:D, :]
    w1j = W1[D:, :]
    scal = jnp.stack([b4[0], W5[0, 0], W5[1, 0], b5[0]])
    return _tc_dense(um, jm, ug, jg, w1u, w1j, b1.reshape(1, -1),
                     W2, b2.reshape(1, -1), W3, b3.reshape(1, -1),
                     W4.reshape(1, -1), scal)


# speedup vs baseline: 2.4198x; 1.0007x over previous
import functools

import jax
import jax.numpy as jnp
from jax import lax
from jax.experimental import pallas as pl
from jax.experimental.pallas import tpu as pltpu
from jax.experimental.pallas import tpu_sc as plsc

B = 16384
D = 32
NC = 2
NS = 16
NW = NC * NS
BPW = B // NW
CHUNK = 128
NCHUNK = BPW // CHUNK
L = 16


@functools.lru_cache(maxsize=None)
def _make_sc_gather():
    mesh = plsc.VectorSubcoreMesh(
        core_axis_name="c", subcore_axis_name="s", num_cores=NC, num_subcores=NS
    )

    @functools.partial(
        pl.kernel,
        out_type=jax.ShapeDtypeStruct((4, B, D), jnp.float32),
        mesh=mesh,
        scratch_types=[
            pltpu.VMEM((BPW,), jnp.int32),
            pltpu.VMEM((BPW,), jnp.int32),
            pltpu.VMEM((CHUNK, D), jnp.float32),
            pltpu.VMEM((CHUNK, D), jnp.float32),
            pltpu.VMEM((CHUNK, D), jnp.float32),
            pltpu.VMEM((CHUNK, D), jnp.float32),
            pltpu.SemaphoreType.DMA,
            pltpu.SemaphoreType.DMA,
        ],
    )
    def _sc_gather(uid_h, jid_h, umt_h, jmt_h, ugt_h, jgt_h, out,
                   uidx, jidx, bum, bjm, bug, bjg, sem, wsem):
        wid = lax.axis_index("s") * NC + lax.axis_index("c")
        base = wid * BPW
        pltpu.sync_copy(uid_h.at[pl.ds(base, BPW)], uidx)
        pltpu.sync_copy(jid_h.at[pl.ds(base, BPW)], jidx)

        for c in range(NCHUNK):
            if c > 0:
                pltpu.make_async_copy(bum, out.at[0, pl.ds(base, CHUNK)], wsem).wait()
                pltpu.make_async_copy(bjm, out.at[1, pl.ds(base, CHUNK)], wsem).wait()
                pltpu.make_async_copy(bug, out.at[2, pl.ds(base, CHUNK)], wsem).wait()
                pltpu.make_async_copy(bjg, out.at[3, pl.ds(base, CHUNK)], wsem).wait()

            def issue(g, _, c=c):
                vu = uidx[pl.ds(c * CHUNK + g * L, L)]
                vj = jidx[pl.ds(c * CHUNK + g * L, L)]
                for i in range(L):
                    u = vu[i]
                    j = vj[i]
                    k = g * L + i
                    pltpu.async_copy(umt_h.at[pl.ds(u, 1)], bum.at[pl.ds(k, 1)], sem)
                    pltpu.async_copy(jmt_h.at[pl.ds(j, 1)], bjm.at[pl.ds(k, 1)], sem)
                    pltpu.async_copy(ugt_h.at[pl.ds(u, 1)], bug.at[pl.ds(k, 1)], sem)
                    pltpu.async_copy(jgt_h.at[pl.ds(j, 1)], bjg.at[pl.ds(k, 1)], sem)
                return ()
            lax.fori_loop(0, CHUNK // L, issue, ())

            def drain(k, _):
                pltpu.make_async_copy(umt_h.at[pl.ds(0, 1)], bum.at[pl.ds(0, 1)], sem).wait()
                pltpu.make_async_copy(jmt_h.at[pl.ds(0, 1)], bjm.at[pl.ds(0, 1)], sem).wait()
                pltpu.make_async_copy(ugt_h.at[pl.ds(0, 1)], bug.at[pl.ds(0, 1)], sem).wait()
                pltpu.make_async_copy(jgt_h.at[pl.ds(0, 1)], bjg.at[pl.ds(0, 1)], sem).wait()
                return ()
            lax.fori_loop(0, CHUNK, drain, ())

            dst = pl.ds(base + c * CHUNK, CHUNK)
            pltpu.async_copy(bum, out.at[0, dst], wsem)
            pltpu.async_copy(bjm, out.at[1, dst], wsem)
            pltpu.async_copy(bug, out.at[2, dst], wsem)
            pltpu.async_copy(bjg, out.at[3, dst], wsem)

        pltpu.make_async_copy(bum, out.at[0, pl.ds(base, CHUNK)], wsem).wait()
        pltpu.make_async_copy(bjm, out.at[1, pl.ds(base, CHUNK)], wsem).wait()
        pltpu.make_async_copy(bug, out.at[2, pl.ds(base, CHUNK)], wsem).wait()
        pltpu.make_async_copy(bjg, out.at[3, pl.ds(base, CHUNK)], wsem).wait()

    return _sc_gather


BLK = 2048


def _tc_body(g, w1, b1, w2, b2, w3, b3, w4, w5, b4, b5, out):
    um = g[0]
    jm = g[1]
    ug = g[2]
    jg = g[3]
    w1v = w1[:]
    x = jnp.maximum(um @ w1v[:D, :] + jm @ w1v[D:, :] + b1[:], 0.0)
    x = jnp.maximum(x @ w2[:] + b2[:], 0.0)
    x = jnp.maximum(x @ w3[:] + b3[:], 0.0)
    x = jnp.maximum(x @ w4[:] + b4[0], 0.0)
    dot = jnp.sum(ug * jg, axis=1, keepdims=True)
    su = jnp.sum(ug * ug, axis=1, keepdims=True)
    sj = jnp.sum(jg * jg, axis=1, keepdims=True)
    gmf = dot * lax.rsqrt(jnp.maximum(su, 1e-12)) * lax.rsqrt(jnp.maximum(sj, 1e-12))
    out[:] = x * w5[0, 0] + gmf * w5[1, 0] + b5[0]


def _tc_dense(g, w1, b1, w2, b2, w3, b3, w4, w5, b4, b5):
    full = lambda a: pl.BlockSpec(a.shape, lambda i, _n=a.ndim: (0,) * _n)
    smem = pl.BlockSpec(memory_space=pltpu.SMEM)
    return pl.pallas_call(
        _tc_body,
        grid=(B // BLK,),
        in_specs=[pl.BlockSpec((4, BLK, D), lambda i: (0, i, 0)),
                  full(w1), full(b1), full(w2), full(b2), full(w3), full(b3),
                  full(w4), smem, smem, smem],
        out_specs=pl.BlockSpec((BLK, 1), lambda i: (i, 0)),
        out_shape=jax.ShapeDtypeStruct((B, 1), jnp.float32),
    )(g, w1, b1, w2, b2, w3, b3, w4, w5, b4, b5)


def kernel(user_ids, joke_ids, user_mlp_table, joke_mlp_table,
           user_gmf_table, joke_gmf_table,
           W1, b1, W2, b2, W3, b3, W4, b4, W5, b5):
    uid = user_ids.astype(jnp.int32)
    jid = joke_ids.astype(jnp.int32)
    g = _make_sc_gather()(uid, jid, user_mlp_table, joke_mlp_table,
                          user_gmf_table, joke_gmf_table)
    return _tc_dense(g, W1, b1, W2, b2, W3, b3, W4, W5, b4, b5)


# speedup vs baseline: 2.4209x; 1.0004x over previous
import functools

import jax
import jax.numpy as jnp
from jax import lax
from jax.experimental import pallas as pl
from jax.experimental.pallas import tpu as pltpu
from jax.experimental.pallas import tpu_sc as plsc

B = 16384
D = 32
NC = 2
NS = 16
NW = NC * NS
BPW = B // NW
CHUNK = 128
NCHUNK = BPW // CHUNK
L = 16


@functools.lru_cache(maxsize=None)
def _make_sc_gather():
    mesh = plsc.VectorSubcoreMesh(
        core_axis_name="c", subcore_axis_name="s", num_cores=NC, num_subcores=NS
    )

    @functools.partial(
        pl.kernel,
        out_type=jax.ShapeDtypeStruct((4, B, D), jnp.float32),
        mesh=mesh,
        scratch_types=[
            pltpu.VMEM((BPW,), jnp.int32),
            pltpu.VMEM((BPW,), jnp.int32),
            pltpu.VMEM((CHUNK, D), jnp.float32),
            pltpu.VMEM((CHUNK, D), jnp.float32),
            pltpu.VMEM((CHUNK, D), jnp.float32),
            pltpu.VMEM((CHUNK, D), jnp.float32),
            pltpu.SemaphoreType.DMA,
            pltpu.SemaphoreType.DMA,
        ],
        compiler_params=pltpu.CompilerParams(skip_device_barrier=True),
    )
    def _sc_gather(uid_h, jid_h, umt_h, jmt_h, ugt_h, jgt_h, out,
                   uidx, jidx, bum, bjm, bug, bjg, sem, wsem):
        wid = lax.axis_index("s") * NC + lax.axis_index("c")
        base = wid * BPW
        pltpu.sync_copy(uid_h.at[pl.ds(base, BPW)], uidx)
        pltpu.sync_copy(jid_h.at[pl.ds(base, BPW)], jidx)

        for c in range(NCHUNK):
            if c > 0:
                pltpu.make_async_copy(bum, out.at[0, pl.ds(base, CHUNK)], wsem).wait()
                pltpu.make_async_copy(bjm, out.at[1, pl.ds(base, CHUNK)], wsem).wait()
                pltpu.make_async_copy(bug, out.at[2, pl.ds(base, CHUNK)], wsem).wait()
                pltpu.make_async_copy(bjg, out.at[3, pl.ds(base, CHUNK)], wsem).wait()

            def issue(g, _, c=c):
                vu = uidx[pl.ds(c * CHUNK + g * L, L)]
                vj = jidx[pl.ds(c * CHUNK + g * L, L)]
                for i in range(L):
                    u = vu[i]
                    j = vj[i]
                    k = g * L + i
                    pltpu.async_copy(umt_h.at[pl.ds(u, 1)], bum.at[pl.ds(k, 1)], sem)
                    pltpu.async_copy(jmt_h.at[pl.ds(j, 1)], bjm.at[pl.ds(k, 1)], sem)
                    pltpu.async_copy(ugt_h.at[pl.ds(u, 1)], bug.at[pl.ds(k, 1)], sem)
                    pltpu.async_copy(jgt_h.at[pl.ds(j, 1)], bjg.at[pl.ds(k, 1)], sem)
                return ()
            lax.fori_loop(0, CHUNK // L, issue, ())

            def drain(k, _):
                pltpu.make_async_copy(umt_h.at[pl.ds(0, 1)], bum.at[pl.ds(0, 1)], sem).wait()
                pltpu.make_async_copy(jmt_h.at[pl.ds(0, 1)], bjm.at[pl.ds(0, 1)], sem).wait()
                pltpu.make_async_copy(ugt_h.at[pl.ds(0, 1)], bug.at[pl.ds(0, 1)], sem).wait()
                pltpu.make_async_copy(jgt_h.at[pl.ds(0, 1)], bjg.at[pl.ds(0, 1)], sem).wait()
                return ()
            lax.fori_loop(0, CHUNK, drain, ())

            dst = pl.ds(base + c * CHUNK, CHUNK)
            pltpu.async_copy(bum, out.at[0, dst], wsem)
            pltpu.async_copy(bjm, out.at[1, dst], wsem)
            pltpu.async_copy(bug, out.at[2, dst], wsem)
            pltpu.async_copy(bjg, out.at[3, dst], wsem)

        pltpu.make_async_copy(bum, out.at[0, pl.ds(base, CHUNK)], wsem).wait()
        pltpu.make_async_copy(bjm, out.at[1, pl.ds(base, CHUNK)], wsem).wait()
        pltpu.make_async_copy(bug, out.at[2, pl.ds(base, CHUNK)], wsem).wait()
        pltpu.make_async_copy(bjg, out.at[3, pl.ds(base, CHUNK)], wsem).wait()

    return _sc_gather


BLK = 2048


def _tc_body(g, w1, b1, w2, b2, w3, b3, w4, w5, b4, b5, out):
    um = g[0]
    jm = g[1]
    ug = g[2]
    jg = g[3]
    w1v = w1[:]
    x = jnp.maximum(um @ w1v[:D, :] + jm @ w1v[D:, :] + b1[:], 0.0)
    x = jnp.maximum(x @ w2[:] + b2[:], 0.0)
    x = jnp.maximum(x @ w3[:] + b3[:], 0.0)
    x = jnp.maximum(x @ w4[:] + b4[0], 0.0)
    dot = jnp.sum(ug * jg, axis=1, keepdims=True)
    su = jnp.sum(ug * ug, axis=1, keepdims=True)
    sj = jnp.sum(jg * jg, axis=1, keepdims=True)
    gmf = dot * lax.rsqrt(jnp.maximum(su, 1e-12)) * lax.rsqrt(jnp.maximum(sj, 1e-12))
    out[:] = x * w5[0, 0] + gmf * w5[1, 0] + b5[0]


def _tc_dense(g, w1, b1, w2, b2, w3, b3, w4, w5, b4, b5):
    full = lambda a: pl.BlockSpec(a.shape, lambda i, _n=a.ndim: (0,) * _n)
    smem = pl.BlockSpec(memory_space=pltpu.SMEM)
    return pl.pallas_call(
        _tc_body,
        grid=(B // BLK,),
        in_specs=[pl.BlockSpec((4, BLK, D), lambda i: (0, i, 0)),
                  full(w1), full(b1), full(w2), full(b2), full(w3), full(b3),
                  full(w4), smem, smem, smem],
        out_specs=pl.BlockSpec((BLK, 1), lambda i: (i, 0)),
        out_shape=jax.ShapeDtypeStruct((B, 1), jnp.float32),
    )(g, w1, b1, w2, b2, w3, b3, w4, w5, b4, b5)


def kernel(user_ids, joke_ids, user_mlp_table, joke_mlp_table,
           user_gmf_table, joke_gmf_table,
           W1, b1, W2, b2, W3, b3, W4, b4, W5, b5):
    uid = user_ids.astype(jnp.int32)
    jid = joke_ids.astype(jnp.int32)
    g = _make_sc_gather()(uid, jid, user_mlp_table, joke_mlp_table,
                          user_gmf_table, joke_gmf_table)
    return _tc_dense(g, W1, b1, W2, b2, W3, b3, W4, W5, b4, b5)
